# baseline (speedup 1.0000x reference)
import functools

import jax
import jax.numpy as jnp
from jax.experimental import pallas as pl
from jax.experimental.pallas import tpu as pltpu

_H_PHASOR = 192
_DELTA0 = 7.0
_TOPK_FEAT = 32
_DT = 0.001
_TAU = 0.02


def _gate_logits(x, gate_W, gate_b):
    q = jnp.mean(x, axis=-1)
    freqs = _DELTA0 * jnp.arange(1, _H_PHASOR + 1, dtype=jnp.float32)
    ang = q[:, None] * freqs[None, :]
    tf = jnp.concatenate([jnp.cos(ang), jnp.sin(ang)], axis=-1)
    s0 = jnp.mean(tf, axis=-1)
    w = jnp.exp(-jnp.arange(_TOPK_FEAT, dtype=jnp.float32) * _DT / _TAU)
    s1 = jnp.full_like(q, jnp.sum(w) / x.shape[-1])
    gate_inputs = jnp.stack([s0, s1], axis=-1)
    return gate_inputs @ gate_W + gate_b


def _moe_kernel(gl_ref, x_ref, w1_ref, b1_ref, w2_ref, b2_ref, out_ref, *,
                e_num, f):
    tb, d = x_ref.shape

    gl = gl_ref[...]
    m = jnp.max(gl, axis=1, keepdims=True)
    eg = jnp.exp(gl - m)
    p = eg / jnp.sum(eg, axis=1, keepdims=True)

    col = jax.lax.broadcasted_iota(jnp.int32, (tb, e_num), 1)
    i1 = jnp.min(jnp.where(gl == m, col, e_num), axis=1, keepdims=True)
    mask1 = col == i1
    gl2 = jnp.where(mask1, -jnp.inf, gl)
    m2 = jnp.max(gl2, axis=1, keepdims=True)
    i2 = jnp.min(jnp.where(gl2 == m2, col, e_num), axis=1, keepdims=True)
    mask = mask1 | (col == i2)

    gated = jnp.where(mask, p, 0.0)
    g = gated / (jnp.sum(gated, axis=1, keepdims=True) + 1e-9)

    lane = jax.lax.broadcasted_iota(jnp.int32, (e_num, e_num * f), 1)
    row = jax.lax.broadcasted_iota(jnp.int32, (e_num, e_num * f), 0)
    expand = (lane // f == row).astype(jnp.float32)
    scale = jnp.dot(g, expand, preferred_element_type=jnp.float32)

    xb = x_ref[...]
    h = jnp.dot(xb, w1_ref[...], preferred_element_type=jnp.float32)
    h = jax.nn.gelu(h + b1_ref[...])
    hb = (h * scale).astype(jnp.bfloat16)
    out = jnp.dot(hb, w2_ref[...], preferred_element_type=jnp.float32)
    out_ref[...] = out + jnp.dot(g, b2_ref[...],
                                 preferred_element_type=jnp.float32)


@jax.jit
def kernel(query_embedding, gate_W, gate_b, W1, b1, W2, b2):
    x = query_embedding
    b_sz, d = x.shape
    e_num, _, f = W1.shape

    gl = _gate_logits(x, gate_W, gate_b)
    x16 = x.astype(jnp.bfloat16)
    w1s = jnp.transpose(W1, (1, 0, 2)).reshape(d, e_num * f).astype(jnp.bfloat16)
    w2s = W2.reshape(e_num * f, d).astype(jnp.bfloat16)
    b1s = b1.reshape(1, e_num * f)

    tb = 256 if b_sz % 256 == 0 else b_sz
    grid = (b_sz // tb,)

    out = pl.pallas_call(
        functools.partial(_moe_kernel, e_num=e_num, f=f),
        grid=grid,
        in_specs=[
            pl.BlockSpec((tb, e_num), lambda i: (i, 0)),
            pl.BlockSpec((tb, d), lambda i: (i, 0)),
            pl.BlockSpec((d, e_num * f), lambda i: (0, 0)),
            pl.BlockSpec((1, e_num * f), lambda i: (0, 0)),
            pl.BlockSpec((e_num * f, d), lambda i: (0, 0)),
            pl.BlockSpec((e_num, d), lambda i: (0, 0)),
        ],
        out_specs=pl.BlockSpec((tb, d), lambda i: (i, 0)),
        out_shape=jax.ShapeDtypeStruct((b_sz, d), jnp.float32),
        compiler_params=pltpu.CompilerParams(
            dimension_semantics=("arbitrary",),
        ),
    )(gl, x16, w1s, b1s, w2s, b2)
    return out


# speedup vs baseline: 1.0801x; 1.0801x over previous
import functools

import jax
import jax.numpy as jnp
from jax.experimental import pallas as pl
from jax.experimental.pallas import tpu as pltpu

_H_PHASOR = 192
_DELTA0 = 7.0
_TOPK_FEAT = 32
_DT = 0.001
_TAU = 0.02


def _gate_logits(x, gate_W, gate_b):
    q = jnp.mean(x, axis=-1)
    freqs = _DELTA0 * jnp.arange(1, _H_PHASOR + 1, dtype=jnp.float32)
    ang = q[:, None] * freqs[None, :]
    tf = jnp.concatenate([jnp.cos(ang), jnp.sin(ang)], axis=-1)
    s0 = jnp.mean(tf, axis=-1)
    w = jnp.exp(-jnp.arange(_TOPK_FEAT, dtype=jnp.float32) * _DT / _TAU)
    s1 = jnp.full_like(q, jnp.sum(w) / x.shape[-1])
    gate_inputs = jnp.stack([s0, s1], axis=-1)
    return gate_inputs @ gate_W + gate_b


def _cast_kernel(w1_ref, w2_ref, o1_ref, o2_ref):
    o1_ref[...] = w1_ref[...].astype(jnp.bfloat16)
    o2_ref[...] = w2_ref[...].astype(jnp.bfloat16)


def _cast_weights(W1, W2):
    e_num, d, f = W1.shape
    return pl.pallas_call(
        _cast_kernel,
        grid=(e_num,),
        in_specs=[
            pl.BlockSpec((1, d, f), lambda e: (e, 0, 0)),
            pl.BlockSpec((1, f, d), lambda e: (e, 0, 0)),
        ],
        out_specs=[
            pl.BlockSpec((1, d, f), lambda e: (e, 0, 0)),
            pl.BlockSpec((1, f, d), lambda e: (e, 0, 0)),
        ],
        out_shape=[
            jax.ShapeDtypeStruct((e_num, d, f), jnp.bfloat16),
            jax.ShapeDtypeStruct((e_num, f, d), jnp.bfloat16),
        ],
        compiler_params=pltpu.CompilerParams(
            dimension_semantics=("arbitrary",),
        ),
    )(W1, W2)


def _moe_kernel(gl_ref, x_ref, w1_ref, b1_ref, w2_ref, b2_ref, out_ref):
    tb, d = x_ref.shape
    e_num = gl_ref.shape[1]

    gl = gl_ref[...]
    m = jnp.max(gl, axis=1, keepdims=True)
    eg = jnp.exp(gl - m)
    p = eg / jnp.sum(eg, axis=1, keepdims=True)

    col = jax.lax.broadcasted_iota(jnp.int32, (tb, e_num), 1)
    i1 = jnp.min(jnp.where(gl == m, col, e_num), axis=1, keepdims=True)
    mask1 = col == i1
    gl2 = jnp.where(mask1, -jnp.inf, gl)
    m2 = jnp.max(gl2, axis=1, keepdims=True)
    i2 = jnp.min(jnp.where(gl2 == m2, col, e_num), axis=1, keepdims=True)
    mask = mask1 | (col == i2)

    gated = jnp.where(mask, p, 0.0)
    g = gated / (jnp.sum(gated, axis=1, keepdims=True) + 1e-9)

    xb = x_ref[...].astype(jnp.bfloat16)
    acc = jnp.zeros((tb, d), jnp.float32)
    for e in range(e_num):
        h = jnp.dot(xb, w1_ref[e], preferred_element_type=jnp.float32)
        h = jax.nn.gelu(h + b1_ref[e][None, :])
        o = jnp.dot(h.astype(jnp.bfloat16), w2_ref[e],
                    preferred_element_type=jnp.float32)
        o = o + b2_ref[e][None, :]
        acc = acc + g[:, e:e + 1] * o
    out_ref[...] = acc


@jax.jit
def kernel(query_embedding, gate_W, gate_b, W1, b1, W2, b2):
    x = query_embedding
    b_sz, d = x.shape
    e_num, _, f = W1.shape

    gl = _gate_logits(x, gate_W, gate_b)
    w1, w2 = _cast_weights(W1, W2)

    tb = 256 if b_sz % 256 == 0 else b_sz
    grid = (b_sz // tb,)

    out = pl.pallas_call(
        _moe_kernel,
        grid=grid,
        in_specs=[
            pl.BlockSpec((tb, e_num), lambda i: (i, 0)),
            pl.BlockSpec((tb, d), lambda i: (i, 0)),
            pl.BlockSpec((e_num, d, f), lambda i: (0, 0, 0)),
            pl.BlockSpec((e_num, f), lambda i: (0, 0)),
            pl.BlockSpec((e_num, f, d), lambda i: (0, 0, 0)),
            pl.BlockSpec((e_num, d), lambda i: (0, 0)),
        ],
        out_specs=pl.BlockSpec((tb, d), lambda i: (i, 0)),
        out_shape=jax.ShapeDtypeStruct((b_sz, d), jnp.float32),
        compiler_params=pltpu.CompilerParams(
            dimension_semantics=("arbitrary",),
        ),
    )(gl, x, w1, b1, w2, b2)
    return out


# speedup vs baseline: 1.1271x; 1.0435x over previous
import functools

import jax
import jax.numpy as jnp
from jax.experimental import pallas as pl
from jax.experimental.pallas import tpu as pltpu

_H_PHASOR = 192
_DELTA0 = 7.0
_TOPK_FEAT = 32
_DT = 0.001
_TAU = 0.02


def _gate_logits(x, gate_W, gate_b):
    q = jnp.mean(x, axis=-1)
    freqs = _DELTA0 * jnp.arange(1, _H_PHASOR + 1, dtype=jnp.float32)
    ang = q[:, None] * freqs[None, :]
    tf = jnp.concatenate([jnp.cos(ang), jnp.sin(ang)], axis=-1)
    s0 = jnp.mean(tf, axis=-1)
    w = jnp.exp(-jnp.arange(_TOPK_FEAT, dtype=jnp.float32) * _DT / _TAU)
    s1 = jnp.full_like(q, jnp.sum(w) / x.shape[-1])
    gate_inputs = jnp.stack([s0, s1], axis=-1)
    return gate_inputs @ gate_W + gate_b


def _cast_kernel(w1_ref, w2_ref, o1_ref, o2_ref):
    o1_ref[...] = w1_ref[0].astype(jnp.bfloat16)
    o2_ref[...] = w2_ref[0].astype(jnp.bfloat16)


def _cast_weights(W1, W2):
    e_num, d, f = W1.shape
    return pl.pallas_call(
        _cast_kernel,
        grid=(e_num,),
        in_specs=[
            pl.BlockSpec((1, d, f), lambda e: (e, 0, 0)),
            pl.BlockSpec((1, f, d), lambda e: (e, 0, 0)),
        ],
        out_specs=[
            pl.BlockSpec((d, f), lambda e: (0, e)),
            pl.BlockSpec((f, d), lambda e: (e, 0)),
        ],
        out_shape=[
            jax.ShapeDtypeStruct((d, e_num * f), jnp.bfloat16),
            jax.ShapeDtypeStruct((e_num * f, d), jnp.bfloat16),
        ],
        compiler_params=pltpu.CompilerParams(
            dimension_semantics=("arbitrary",),
        ),
    )(W1, W2)


def _moe_kernel(gl_ref, x_ref, w1_ref, b1_ref, w2_ref, b2_ref, out_ref, *,
                e_num, f):
    tb, d = x_ref.shape

    gl = gl_ref[...]
    m = jnp.max(gl, axis=1, keepdims=True)
    eg = jnp.exp(gl - m)
    p = eg / jnp.sum(eg, axis=1, keepdims=True)

    col = jax.lax.broadcasted_iota(jnp.int32, (tb, e_num), 1)
    i1 = jnp.min(jnp.where(gl == m, col, e_num), axis=1, keepdims=True)
    mask1 = col == i1
    gl2 = jnp.where(mask1, -jnp.inf, gl)
    m2 = jnp.max(gl2, axis=1, keepdims=True)
    i2 = jnp.min(jnp.where(gl2 == m2, col, e_num), axis=1, keepdims=True)
    mask = mask1 | (col == i2)

    gated = jnp.where(mask, p, 0.0)
    g = gated / (jnp.sum(gated, axis=1, keepdims=True) + 1e-9)

    lane = jax.lax.broadcasted_iota(jnp.int32, (e_num, e_num * f), 1)
    row = jax.lax.broadcasted_iota(jnp.int32, (e_num, e_num * f), 0)
    expand = (lane // f == row).astype(jnp.float32)
    scale = jnp.dot(g, expand, preferred_element_type=jnp.float32)

    xb = x_ref[...].astype(jnp.bfloat16)
    h = jnp.dot(xb, w1_ref[...], preferred_element_type=jnp.float32)
    h = jax.nn.gelu(h + b1_ref[...])
    hb = (h * scale).astype(jnp.bfloat16)
    out = jnp.dot(hb, w2_ref[...], preferred_element_type=jnp.float32)
    out_ref[...] = out + jnp.dot(g, b2_ref[...],
                                 preferred_element_type=jnp.float32)


@jax.jit
def kernel(query_embedding, gate_W, gate_b, W1, b1, W2, b2):
    x = query_embedding
    b_sz, d = x.shape
    e_num, _, f = W1.shape

    gl = _gate_logits(x, gate_W, gate_b)
    w1s, w2s = _cast_weights(W1, W2)
    b1s = b1.reshape(1, e_num * f)

    tb = 256 if b_sz % 256 == 0 else b_sz
    grid = (b_sz // tb,)

    out = pl.pallas_call(
        functools.partial(_moe_kernel, e_num=e_num, f=f),
        grid=grid,
        in_specs=[
            pl.BlockSpec((tb, e_num), lambda i: (i, 0)),
            pl.BlockSpec((tb, d), lambda i: (i, 0)),
            pl.BlockSpec((d, e_num * f), lambda i: (0, 0)),
            pl.BlockSpec((1, e_num * f), lambda i: (0, 0)),
            pl.BlockSpec((e_num * f, d), lambda i: (0, 0)),
            pl.BlockSpec((e_num, d), lambda i: (0, 0)),
        ],
        out_specs=pl.BlockSpec((tb, d), lambda i: (i, 0)),
        out_shape=jax.ShapeDtypeStruct((b_sz, d), jnp.float32),
        compiler_params=pltpu.CompilerParams(
            dimension_semantics=("arbitrary",),
        ),
    )(gl, x, w1s, b1s, w2s, b2)
    return out


# speedup vs baseline: 1.1574x; 1.0269x over previous
import functools

import jax
import jax.numpy as jnp
from jax.experimental import pallas as pl
from jax.experimental.pallas import tpu as pltpu

_H_PHASOR = 192
_DELTA0 = 7.0
_TOPK_FEAT = 32
_DT = 0.001
_TAU = 0.02


def _gate_logits(x, gate_W, gate_b):
    q = jnp.mean(x, axis=-1)
    freqs = _DELTA0 * jnp.arange(1, _H_PHASOR + 1, dtype=jnp.float32)
    ang = q[:, None] * freqs[None, :]
    tf = jnp.concatenate([jnp.cos(ang), jnp.sin(ang)], axis=-1)
    s0 = jnp.mean(tf, axis=-1)
    w = jnp.exp(-jnp.arange(_TOPK_FEAT, dtype=jnp.float32) * _DT / _TAU)
    s1 = jnp.full_like(q, jnp.sum(w) / x.shape[-1])
    gate_inputs = jnp.stack([s0, s1], axis=-1)
    return gate_inputs @ gate_W + gate_b


def _cast_kernel(w1_ref, w2_ref, o1_ref, o2_ref):
    o1_ref[...] = w1_ref[0].astype(jnp.bfloat16)
    o2_ref[...] = w2_ref[0].astype(jnp.bfloat16)


def _cast_weights(W1, W2):
    e_num, d, f = W1.shape
    return pl.pallas_call(
        _cast_kernel,
        grid=(e_num,),
        in_specs=[
            pl.BlockSpec((1, d, f), lambda e: (e, 0, 0)),
            pl.BlockSpec((1, f, d), lambda e: (e, 0, 0)),
        ],
        out_specs=[
            pl.BlockSpec((d, f), lambda e: (0, e)),
            pl.BlockSpec((f, d), lambda e: (e, 0)),
        ],
        out_shape=[
            jax.ShapeDtypeStruct((d, e_num * f), jnp.bfloat16),
            jax.ShapeDtypeStruct((e_num * f, d), jnp.bfloat16),
        ],
        compiler_params=pltpu.CompilerParams(
            dimension_semantics=("arbitrary",),
        ),
    )(W1, W2)


def _moe_kernel(gl_ref, x_ref, w1_ref, b1_ref, w2_ref, b2_ref, out_ref, *,
                e_num, f):
    tb, d = x_ref.shape

    gl = gl_ref[...]
    m = jnp.max(gl, axis=1, keepdims=True)
    eg = jnp.exp(gl - m)
    p = eg / jnp.sum(eg, axis=1, keepdims=True)

    col = jax.lax.broadcasted_iota(jnp.int32, (tb, e_num), 1)
    i1 = jnp.min(jnp.where(gl == m, col, e_num), axis=1, keepdims=True)
    mask1 = col == i1
    gl2 = jnp.where(mask1, -jnp.inf, gl)
    m2 = jnp.max(gl2, axis=1, keepdims=True)
    i2 = jnp.min(jnp.where(gl2 == m2, col, e_num), axis=1, keepdims=True)
    mask = mask1 | (col == i2)

    gated = jnp.where(mask, p, 0.0)
    g = gated / (jnp.sum(gated, axis=1, keepdims=True) + 1e-9)

    lane = jax.lax.broadcasted_iota(jnp.int32, (e_num, e_num * f), 1)
    row = jax.lax.broadcasted_iota(jnp.int32, (e_num, e_num * f), 0)
    expand = (lane // f == row).astype(jnp.float32)
    scale = jnp.dot(g, expand, preferred_element_type=jnp.float32)

    xb = x_ref[...].astype(jnp.bfloat16)
    h = jnp.dot(xb, w1_ref[...], preferred_element_type=jnp.float32)
    h = jax.nn.gelu(h + b1_ref[...])
    hb = (h * scale).astype(jnp.bfloat16)
    out = jnp.dot(hb, w2_ref[...], preferred_element_type=jnp.float32)
    out_ref[...] = out + jnp.dot(g, b2_ref[...],
                                 preferred_element_type=jnp.float32)


@jax.jit
def kernel(query_embedding, gate_W, gate_b, W1, b1, W2, b2):
    x = query_embedding
    b_sz, d = x.shape
    e_num, _, f = W1.shape

    gl = _gate_logits(x, gate_W, gate_b)
    w1s, w2s = _cast_weights(W1, W2)
    b1s = b1.reshape(1, e_num * f)

    tb = 512 if b_sz % 512 == 0 else b_sz
    grid = (b_sz // tb,)

    out = pl.pallas_call(
        functools.partial(_moe_kernel, e_num=e_num, f=f),
        grid=grid,
        in_specs=[
            pl.BlockSpec((tb, e_num), lambda i: (i, 0)),
            pl.BlockSpec((tb, d), lambda i: (i, 0)),
            pl.BlockSpec((d, e_num * f), lambda i: (0, 0)),
            pl.BlockSpec((1, e_num * f), lambda i: (0, 0)),
            pl.BlockSpec((e_num * f, d), lambda i: (0, 0)),
            pl.BlockSpec((e_num, d), lambda i: (0, 0)),
        ],
        out_specs=pl.BlockSpec((tb, d), lambda i: (i, 0)),
        out_shape=jax.ShapeDtypeStruct((b_sz, d), jnp.float32),
        compiler_params=pltpu.CompilerParams(
            dimension_semantics=("arbitrary",),
            vmem_limit_bytes=100 * 1024 * 1024,
        ),
    )(gl, x, w1s, b1s, w2s, b2)
    return out
